# baseline (speedup 1.0000x reference)
import functools

import jax
import jax.numpy as jnp
import numpy as np
from jax import lax
from jax.experimental import pallas as pl
from jax.experimental.pallas import tpu as pltpu
from jax.experimental.pallas import tpu_sc as plsc

B, N, D = 16, 1024, 512
T = B * N
K = 8
CD = 64
V = 512

NC, NS = 2, 16
NW = NC * NS
TPW = T // NW
NG = TPW // 16

NPAIR = K * (K + 1) // 2
BS = V // 2

ROWS = 512


def _topk_body(x_ref, cb_ref, idx_ref, g_ref):
    x = x_ref[...]
    col_f = lax.broadcasted_iota(jnp.int32, (ROWS, D), 1).astype(jnp.float32)
    neg_inf = jnp.float32(-jnp.inf)
    sels = []
    for t in range(K):
        m = jnp.max(x, axis=1, keepdims=True)
        cand = jnp.where(x == m, col_f, jnp.float32(1e9))
        sel_f = jnp.min(cand, axis=1, keepdims=True)
        sels.append(sel_f)
        x = jnp.where(col_f == sel_f, neg_inf, x)
    idx_ref[...] = jnp.concatenate(sels, axis=1).astype(jnp.int32)

    @pl.when(pl.program_id(0) == 0)
    def _():
        c = cb_ref[...]
        nrm = jnp.sqrt(jnp.sum(c * c, axis=1, keepdims=True))
        cn = c / jnp.maximum(nrm, jnp.float32(1e-12))
        cnb = cn.astype(jnp.bfloat16)
        g_ref[...] = lax.dot_general(
            cnb, cnb, (((1,), (1,)), ((), ())),
            preferred_element_type=jnp.float32)


_topk_call = pl.pallas_call(
    _topk_body,
    grid=(T // ROWS,),
    in_specs=[
        pl.BlockSpec((ROWS, D), lambda i: (i, 0)),
        pl.BlockSpec((V, CD), lambda i: (0, 0)),
    ],
    out_specs=[
        pl.BlockSpec((ROWS, K), lambda i: (i, 0)),
        pl.BlockSpec((V, V), lambda i: (0, 0)),
    ],
    out_shape=[
        jax.ShapeDtypeStruct((T, K), jnp.int32),
        jax.ShapeDtypeStruct((V, V), jnp.float32),
    ],
)


NBLK = 4
BSZ = V // NBLK


def _sc_pairs_body(g_hbm, idx_hbm, out_hbm, gbuf_v, idx_v, out_v, sem0, sem1):
    wid = lax.axis_index("s") * NC + lax.axis_index("c")
    zero_f = jnp.zeros((16,), jnp.float32)
    sems = (sem0, sem1)
    NTOT = NBLK * NBLK

    def start_dma(bi, b):
        rb = lax.shift_right_logical(bi, 2)
        cb = bi & 3
        pltpu.async_copy(
            g_hbm.at[pl.ds(rb * BSZ, BSZ), pl.ds(cb * BSZ, BSZ)],
            gbuf_v.at[b], sems[b])

    def wait_dma(b):
        pltpu.make_async_copy(
            g_hbm.at[pl.ds(0, BSZ), pl.ds(0, BSZ)],
            gbuf_v.at[b], sems[b]).wait()

    start_dma(jnp.int32(0), 0)
    start_dma(jnp.int32(1), 1)
    with jax.named_scope("sc_load"):
        pltpu.sync_copy(idx_hbm.at[pl.ds(wid * (TPW * K), TPW * K)], idx_v)
        for p in range(NPAIR):
            out_v[p, :] = zero_f

    iota16 = lax.iota(jnp.int32, 16)
    u7 = jnp.uint32(7)
    u127 = jnp.uint32(127)

    def step_body(s, carry):
        for b in range(2):
            bi = s * 2 + b
            with jax.named_scope("sc_wait"):
                wait_dma(b)
            rbu = lax.shift_right_logical(bi, 2).astype(jnp.uint32)
            cbu = (bi & 3).astype(jnp.uint32)
            cur = gbuf_v.at[b]

            def group_body(g, carry2, cur=cur, rbu=rbu, cbu=cbu):
                gbase = iota16 * K + g * (16 * K)
                rows = [plsc.bitcast(plsc.load_gather(idx_v, [gbase + k]),
                                     jnp.uint32) for k in range(K)]
                his = [lax.shift_right_logical(r, u7) for r in rows]
                los = [plsc.bitcast(r & u127, jnp.int32) for r in rows]
                p = 0
                for k in range(K):
                    for j in range(k, K):
                        m = (his[k] == rbu) & (his[j] == cbu)
                        val = plsc.load_gather(cur, [los[k], los[j]], mask=m)
                        plsc.addupdate(out_v.at[p, :],
                                       jnp.where(m, val, zero_f))
                        p += 1
                return carry2

            with jax.named_scope("sc_accum"):
                lax.fori_loop(0, NG, group_body, 0)

            @pl.when(bi + 2 < NTOT)
            def _():
                start_dma(bi + 2, b)
        return carry

    lax.fori_loop(0, NTOT // 2, step_body, 0)

    with jax.named_scope("sc_out"):
        pltpu.sync_copy(out_v, out_hbm.at[wid])


@functools.cache
def _sc_pairs_call():
    return pl.kernel(
        _sc_pairs_body,
        out_type=jax.ShapeDtypeStruct((NW, NPAIR, 16), jnp.float32),
        mesh=plsc.VectorSubcoreMesh(core_axis_name="c", subcore_axis_name="s"),
        compiler_params=pltpu.CompilerParams(needs_layout_passes=False),
        scratch_types=[
            pltpu.VMEM((2, BSZ, BSZ), jnp.float32),
            pltpu.VMEM((TPW * K,), jnp.int32),
            pltpu.VMEM((NPAIR, 16), jnp.float32),
            pltpu.SemaphoreType.DMA,
            pltpu.SemaphoreType.DMA,
        ],
    )


def _loss_body(p_ref, w_ref, b_ref, o_ref):
    s = jnp.sum(jnp.sum(p_ref[...], axis=2), axis=0)
    mean = s * jnp.float32(1.0 / T)
    o_ref[...] = jnp.sum(jnp.abs(mean - b_ref[...][0]) * w_ref[...][0]).reshape(1, 1)


_loss_call = pl.pallas_call(
    _loss_body,
    out_shape=jax.ShapeDtypeStruct((1, 1), jnp.float32),
)

_W_NP = np.zeros((1, NPAIR), np.float32)
_B_NP = np.zeros((1, NPAIR), np.float32)
_p = 0
for _k in range(K):
    for _j in range(_k, K):
        _W_NP[0, _p] = 1.0 if _j == _k else 2.0
        _B_NP[0, _p] = 1.0 if _j == _k else 0.0
        _p += 1


def kernel(pose_code, codebook):
    pose_flat = pose_code.reshape(T, D)
    idx, gram = _topk_call(pose_flat, codebook)
    partials = _sc_pairs_call()(gram, idx.reshape(-1))
    loss2d = _loss_call(partials, jnp.asarray(_W_NP), jnp.asarray(_B_NP))
    return loss2d[0, 0]


# speedup vs baseline: 1.6556x; 1.6556x over previous
import functools

import jax
import jax.numpy as jnp
import numpy as np
from jax import lax
from jax.experimental import pallas as pl
from jax.experimental.pallas import tpu as pltpu
from jax.experimental.pallas import tpu_sc as plsc

B, N, D = 16, 1024, 512
T = B * N
K = 8
CD = 64
V = 512

NC, NS = 2, 16
NW = NC * NS
TPW = T // NW
NG = TPW // 16

NPAIR = K * (K + 1) // 2
BS = V // 2

ROWS = 512


def _topk_body(x_ref, cb_ref, idx_ref, g_ref):
    x = x_ref[...]
    col_f = lax.broadcasted_iota(jnp.int32, (ROWS, D), 1).astype(jnp.float32)
    neg_inf = jnp.float32(-jnp.inf)
    sels = []
    for t in range(K):
        m = jnp.max(x, axis=1, keepdims=True)
        cand = jnp.where(x == m, col_f, jnp.float32(1e9))
        sel_f = jnp.min(cand, axis=1, keepdims=True)
        sels.append(sel_f)
        x = jnp.where(col_f == sel_f, neg_inf, x)
    idx_ref[...] = jnp.concatenate(sels, axis=1).astype(jnp.int32)

    @pl.when(pl.program_id(0) == 0)
    def _():
        c = cb_ref[...]
        nrm = jnp.sqrt(jnp.sum(c * c, axis=1, keepdims=True))
        cn = c / jnp.maximum(nrm, jnp.float32(1e-12))
        cnb = cn.astype(jnp.bfloat16)
        g_ref[...] = lax.dot_general(
            cnb, cnb, (((1,), (1,)), ((), ())),
            preferred_element_type=jnp.float32)


_topk_call = pl.pallas_call(
    _topk_body,
    grid=(T // ROWS,),
    in_specs=[
        pl.BlockSpec((ROWS, D), lambda i: (i, 0)),
        pl.BlockSpec((V, CD), lambda i: (0, 0)),
    ],
    out_specs=[
        pl.BlockSpec((ROWS, K), lambda i: (i, 0)),
        pl.BlockSpec((V, V), lambda i: (0, 0)),
    ],
    out_shape=[
        jax.ShapeDtypeStruct((T, K), jnp.int32),
        jax.ShapeDtypeStruct((V, V), jnp.float32),
    ],
)


def _umin255(u):
    return plsc.bitcast(jnp.minimum(u, jnp.uint32(255)), jnp.int32)


def _sc_pairs_body(g_hbm, idx_hbm, out_hbm, g_sh, gblk_v, idx_v, out_v, sem):
    wid = lax.axis_index("s") * NC + lax.axis_index("c")
    tid = lax.axis_index("s")
    zero_f = jnp.zeros((16,), jnp.float32)
    with jax.named_scope("sc_stage"):
        @pl.when(tid == 0)
        def _():
            pltpu.sync_copy(g_hbm, g_sh)
    with jax.named_scope("sc_load"):
        pltpu.sync_copy(idx_hbm.at[pl.ds(wid * (TPW * K), TPW * K)], idx_v)
        for p in range(NPAIR):
            out_v[p, :] = zero_f
        plsc.subcore_barrier()

    iota16 = lax.iota(jnp.int32, 16)
    bs_u = jnp.uint32(BS)

    for rb, cb in ((0, 0), (1, 0), (1, 1)):
        with jax.named_scope(f"sc_gdma_{rb}{cb}"):
            pltpu.sync_copy(
                g_sh.at[pl.ds(rb * BS, BS), pl.ds(cb * BS, BS)], gblk_v)

        def group_body(g, carry, rb=rb, cb=cb):
            gbase = iota16 * K + g * (16 * K)
            rows = [plsc.bitcast(plsc.load_gather(idx_v, [gbase + k]),
                                 jnp.uint32) for k in range(K)]
            p = 0
            for k in range(K):
                for j in range(k, K):
                    if j == k:
                        imin = imax = rows[k]
                    else:
                        imin = jnp.minimum(rows[k], rows[j])
                        imax = jnp.maximum(rows[k], rows[j])
                    if (rb, cb) == (0, 0):
                        m = imax < bs_u
                        lr, lc = imax, imin
                    elif (rb, cb) == (1, 0):
                        m = (imax >= bs_u) & (imin < bs_u)
                        lr, lc = imax - bs_u, imin
                    else:
                        m = imin >= bs_u
                        lr, lc = imax - bs_u, imin - bs_u
                    val = plsc.load_gather(
                        gblk_v, [_umin255(lr), _umin255(lc)], mask=m)
                    plsc.addupdate(out_v.at[p, :], jnp.where(m, val, zero_f))
                    p += 1
            return carry

        with jax.named_scope(f"sc_accum_{rb}{cb}"):
            lax.fori_loop(0, NG, group_body, 0)

    with jax.named_scope("sc_out"):
        pltpu.sync_copy(out_v, out_hbm.at[wid])


@functools.cache
def _sc_pairs_call():
    return pl.kernel(
        _sc_pairs_body,
        out_type=jax.ShapeDtypeStruct((NW, NPAIR, 16), jnp.float32),
        mesh=plsc.VectorSubcoreMesh(core_axis_name="c", subcore_axis_name="s"),
        compiler_params=pltpu.CompilerParams(needs_layout_passes=False),
        scratch_types=[
            pltpu.VMEM_SHARED((V, V), jnp.float32),
            pltpu.VMEM((BS, BS), jnp.float32),
            pltpu.VMEM((TPW * K,), jnp.int32),
            pltpu.VMEM((NPAIR, 16), jnp.float32),
            pltpu.SemaphoreType.DMA,
        ],
    )


def _loss_body(p_ref, w_ref, b_ref, o_ref):
    s = jnp.sum(jnp.sum(p_ref[...], axis=2), axis=0)
    mean = s * jnp.float32(1.0 / T)
    o_ref[...] = jnp.sum(jnp.abs(mean - b_ref[...][0]) * w_ref[...][0]).reshape(1, 1)


_loss_call = pl.pallas_call(
    _loss_body,
    out_shape=jax.ShapeDtypeStruct((1, 1), jnp.float32),
)

_W_NP = np.zeros((1, NPAIR), np.float32)
_B_NP = np.zeros((1, NPAIR), np.float32)
_p = 0
for _k in range(K):
    for _j in range(_k, K):
        _W_NP[0, _p] = 1.0 if _j == _k else 2.0
        _B_NP[0, _p] = 1.0 if _j == _k else 0.0
        _p += 1


def kernel(pose_code, codebook):
    pose_flat = pose_code.reshape(T, D)
    idx, gram = _topk_call(pose_flat, codebook)
    partials = _sc_pairs_call()(gram, idx.reshape(-1))
    loss2d = _loss_call(partials, jnp.asarray(_W_NP), jnp.asarray(_B_NP))
    return loss2d[0, 0]


# speedup vs baseline: 1.7613x; 1.0638x over previous
import functools

import jax
import jax.numpy as jnp
import numpy as np
from jax import lax
from jax.experimental import pallas as pl
from jax.experimental.pallas import tpu as pltpu
from jax.experimental.pallas import tpu_sc as plsc

B, N, D = 16, 1024, 512
T = B * N
K = 8
CD = 64
V = 512

NC, NS = 2, 16
NW = NC * NS
TPW = T // NW
NG = TPW // 16

NPAIR = K * (K + 1) // 2
BS = V // 2

ROWS = 512


def _topk_body(x_ref, cb_ref, iota_ref, idx_ref, g_ref):
    x = x_ref[...]
    col_f = jnp.broadcast_to(iota_ref[...], (ROWS, D))
    neg_inf = jnp.float32(-jnp.inf)
    sels = []
    for t in range(K):
        m = jnp.max(x, axis=1, keepdims=True)
        cand = jnp.where(x == m, col_f, jnp.float32(1e9))
        sel_f = jnp.min(cand, axis=1, keepdims=True)
        sels.append(sel_f)
        x = jnp.where(col_f == sel_f, neg_inf, x)
    idx_ref[...] = jnp.concatenate(sels, axis=1).T.astype(jnp.int32)

    @pl.when(pl.program_id(0) == 0)
    def _():
        c = cb_ref[...]
        nrm = jnp.sqrt(jnp.sum(c * c, axis=1, keepdims=True))
        cn = c / jnp.maximum(nrm, jnp.float32(1e-12))
        cnb = cn.astype(jnp.bfloat16)
        g_ref[...] = lax.dot_general(
            cnb, cnb, (((1,), (1,)), ((), ())),
            preferred_element_type=jnp.float32)


_topk_call = pl.pallas_call(
    _topk_body,
    grid=(T // ROWS,),
    in_specs=[
        pl.BlockSpec((ROWS, D), lambda i: (i, 0)),
        pl.BlockSpec((V, CD), lambda i: (0, 0)),
        pl.BlockSpec((1, D), lambda i: (0, 0)),
    ],
    out_specs=[
        pl.BlockSpec((K, ROWS), lambda i: (0, i)),
        pl.BlockSpec((V, V), lambda i: (0, 0)),
    ],
    out_shape=[
        jax.ShapeDtypeStruct((K, T), jnp.int32),
        jax.ShapeDtypeStruct((V, V), jnp.float32),
    ],
)


def _umin255(u):
    return plsc.bitcast(jnp.minimum(u, jnp.uint32(255)), jnp.int32)


def _sc_pairs_body(g_hbm, idx_hbm, out_hbm, g_sh, gblk_v, idx_v, out_v, sem):
    wid = lax.axis_index("s") * NC + lax.axis_index("c")
    tid = lax.axis_index("s")
    zero_f = jnp.zeros((16,), jnp.float32)
    with jax.named_scope("sc_stage"):
        @pl.when(tid == 0)
        def _():
            pltpu.sync_copy(g_hbm, g_sh)
    with jax.named_scope("sc_load"):
        pltpu.sync_copy(idx_hbm.at[:, pl.ds(wid * TPW, TPW)], idx_v)
        for p in range(NPAIR):
            out_v[p, :] = zero_f
        plsc.subcore_barrier()

    iota16 = lax.iota(jnp.int32, 16)
    kvecs = [jnp.full((16,), k, jnp.int32) for k in range(K)]
    bs_u = jnp.uint32(BS)

    for rb, cb in ((0, 0), (1, 0), (1, 1)):
        with jax.named_scope(f"sc_gdma_{rb}{cb}"):
            pltpu.sync_copy(
                g_sh.at[pl.ds(rb * BS, BS), pl.ds(cb * BS, BS)], gblk_v)

        def group_body(g, carry, rb=rb, cb=cb):
            toks = iota16 + g * 16
            rows = [plsc.bitcast(plsc.load_gather(idx_v, [kvecs[k], toks]),
                                 jnp.uint32) for k in range(K)]
            p = 0
            for k in range(K):
                for j in range(k, K):
                    if j == k:
                        imin = imax = rows[k]
                    else:
                        imin = jnp.minimum(rows[k], rows[j])
                        imax = jnp.maximum(rows[k], rows[j])
                    if (rb, cb) == (0, 0):
                        m = imax < bs_u
                        lr, lc = imax, imin
                    elif (rb, cb) == (1, 0):
                        m = (imax >= bs_u) & (imin < bs_u)
                        lr, lc = imax - bs_u, imin
                    else:
                        m = imin >= bs_u
                        lr, lc = imax - bs_u, imin - bs_u
                    val = plsc.load_gather(
                        gblk_v, [_umin255(lr), _umin255(lc)], mask=m)
                    plsc.addupdate(out_v.at[p, :], jnp.where(m, val, zero_f))
                    p += 1
            return carry

        with jax.named_scope(f"sc_accum_{rb}{cb}"):
            lax.fori_loop(0, NG, group_body, 0)

    with jax.named_scope("sc_out"):
        pltpu.sync_copy(out_v, out_hbm.at[wid])


@functools.cache
def _sc_pairs_call():
    return pl.kernel(
        _sc_pairs_body,
        out_type=jax.ShapeDtypeStruct((NW, NPAIR, 16), jnp.float32),
        mesh=plsc.VectorSubcoreMesh(core_axis_name="c", subcore_axis_name="s"),
        compiler_params=pltpu.CompilerParams(needs_layout_passes=False),
        scratch_types=[
            pltpu.VMEM_SHARED((V, V), jnp.float32),
            pltpu.VMEM((BS, BS), jnp.float32),
            pltpu.VMEM((K, TPW), jnp.int32),
            pltpu.VMEM((NPAIR, 16), jnp.float32),
            pltpu.SemaphoreType.DMA,
        ],
    )


def _loss_body(p_ref, w_ref, b_ref, o_ref):
    s = jnp.sum(jnp.sum(p_ref[...], axis=2), axis=0)
    mean = s * jnp.float32(1.0 / T)
    o_ref[...] = jnp.sum(jnp.abs(mean - b_ref[...][0]) * w_ref[...][0]).reshape(1, 1)


_loss_call = pl.pallas_call(
    _loss_body,
    out_shape=jax.ShapeDtypeStruct((1, 1), jnp.float32),
)

_W_NP = np.zeros((1, NPAIR), np.float32)
_B_NP = np.zeros((1, NPAIR), np.float32)
_p = 0
for _k in range(K):
    for _j in range(_k, K):
        _W_NP[0, _p] = 1.0 if _j == _k else 2.0
        _B_NP[0, _p] = 1.0 if _j == _k else 0.0
        _p += 1


_IOTA_NP = np.arange(D, dtype=np.float32).reshape(1, D)


def kernel(pose_code, codebook):
    pose_flat = pose_code.reshape(T, D)
    idx, gram = _topk_call(pose_flat, codebook, jnp.asarray(_IOTA_NP))
    partials = _sc_pairs_call()(gram, idx)
    loss2d = _loss_call(partials, jnp.asarray(_W_NP), jnp.asarray(_B_NP))
    return loss2d[0, 0]
